# baseline (speedup 1.0000x reference)
import functools

import jax
import jax.numpy as jnp
from jax import lax
from jax.experimental import pallas as pl
from jax.experimental.pallas import tpu as pltpu
from jax.experimental.pallas import tpu_sc as plsc

_CHUNK = 128


@functools.cache
def _make_gather(V, D, B):
    info = plsc.get_sparse_core_info()
    NC, NS = info.num_cores, info.num_subcores
    NW = NC * NS
    b_per_w = B // NW
    n_chunks = b_per_w // _CHUNK
    mesh = plsc.VectorSubcoreMesh(core_axis_name="c", subcore_axis_name="s")

    @functools.partial(
        pl.kernel,
        mesh=mesh,
        out_type=jax.ShapeDtypeStruct((B, D), jnp.float32),
        scratch_types=[
            pltpu.VMEM((n_chunks, _CHUNK), jnp.int32),
            pltpu.VMEM((b_per_w, D), jnp.float32),
            pltpu.SemaphoreType.DMA,
        ],
        compiler_params=pltpu.CompilerParams(use_tc_tiling_on_sc=False),
    )
    def k(idx_hbm, table_hbm, out_hbm, idx_v, rows_v, sem):
        wid = lax.axis_index("s") * NC + lax.axis_index("c")
        base = wid * b_per_w
        pltpu.sync_copy(idx_hbm.at[pl.ds(wid * n_chunks, n_chunks)], idx_v)
        copies = [
            pltpu.async_copy(
                table_hbm.at[idx_v.at[j]],
                rows_v.at[pl.ds(j * _CHUNK, _CHUNK)],
                sem,
            )
            for j in range(n_chunks)
        ]
        for c in copies:
            c.wait()
        pltpu.sync_copy(rows_v, out_hbm.at[pl.ds(base, b_per_w)])

    return k


def kernel(domain_id, weight):
    (B,) = domain_id.shape
    V, D = weight.shape
    idx2d = domain_id.astype(jnp.int32).reshape(B // _CHUNK, _CHUNK)
    return _make_gather(V, D, B)(idx2d, weight)


# speedup vs baseline: 1.0043x; 1.0043x over previous
import functools

import jax
import jax.numpy as jnp
from jax import lax
from jax.experimental import pallas as pl
from jax.experimental.pallas import tpu as pltpu
from jax.experimental.pallas import tpu_sc as plsc

_CHUNK = 128


@functools.cache
def _make_gather(V, D, B):
    info = plsc.get_sparse_core_info()
    NC, NS = info.num_cores, info.num_subcores
    NW = NC * NS
    b_per_w = B // NW
    n_chunks = b_per_w // _CHUNK
    mesh = plsc.VectorSubcoreMesh(core_axis_name="c", subcore_axis_name="s")

    @functools.partial(
        pl.kernel,
        mesh=mesh,
        out_type=jax.ShapeDtypeStruct((B, D), jnp.float32),
        scratch_types=[
            pltpu.VMEM((n_chunks, _CHUNK), jnp.int32),
            pltpu.VMEM((b_per_w, D), jnp.float32),
            pltpu.SemaphoreType.DMA,
            pltpu.SemaphoreType.DMA,
        ],
        compiler_params=pltpu.CompilerParams(use_tc_tiling_on_sc=False),
    )
    def k(idx_hbm, table_hbm, out_hbm, idx_v, rows_v, gsem, wsem):
        wid = lax.axis_index("s") * NC + lax.axis_index("c")
        base = wid * b_per_w
        pltpu.sync_copy(idx_hbm.at[pl.ds(wid * n_chunks, n_chunks)], idx_v)
        gathers = [
            pltpu.async_copy(
                table_hbm.at[idx_v.at[j]],
                rows_v.at[pl.ds(j * _CHUNK, _CHUNK)],
                gsem,
            )
            for j in range(n_chunks)
        ]
        writebacks = []
        for j, g in enumerate(gathers):
            g.wait()
            writebacks.append(
                pltpu.async_copy(
                    rows_v.at[pl.ds(j * _CHUNK, _CHUNK)],
                    out_hbm.at[pl.ds(base + j * _CHUNK, _CHUNK)],
                    wsem,
                )
            )
        for w in writebacks:
            w.wait()

    return k


def kernel(domain_id, weight):
    (B,) = domain_id.shape
    V, D = weight.shape
    idx2d = domain_id.astype(jnp.int32).reshape(B // _CHUNK, _CHUNK)
    return _make_gather(V, D, B)(idx2d, weight)


# speedup vs baseline: 2.2412x; 2.2315x over previous
import functools

import jax
import jax.numpy as jnp
from jax import lax
from jax.experimental import pallas as pl
from jax.experimental.pallas import tpu as pltpu
from jax.experimental.pallas import tpu_sc as plsc

_IDX_CHUNK = 4096
_L = 16


@functools.cache
def _make_gather(V, D, B):
    info = plsc.get_sparse_core_info()
    NC, NS = info.num_cores, info.num_subcores
    NW = NC * NS
    assert D == NW, "one tile per embedding dim"
    n_chunks = B // _IDX_CHUNK
    mesh = plsc.VectorSubcoreMesh(core_axis_name="c", subcore_axis_name="s")

    @functools.partial(
        pl.kernel,
        mesh=mesh,
        out_type=jax.ShapeDtypeStruct((D, B), jnp.float32),
        scratch_types=[
            pltpu.VMEM((V,), jnp.float32),
            pltpu.VMEM((_IDX_CHUNK,), jnp.int32),
            pltpu.VMEM((_IDX_CHUNK,), jnp.float32),
            pltpu.SemaphoreType.DMA,
        ],
        compiler_params=pltpu.CompilerParams(
            use_tc_tiling_on_sc=True, needs_layout_passes=False
        ),
    )
    def k(wt_hbm, idx_hbm, out_hbm, row_v, idx_v, col_v, sem):
        d = lax.axis_index("s") * NC + lax.axis_index("c")
        pltpu.sync_copy(wt_hbm.at[d], row_v)

        def chunk(c, _):
            pltpu.sync_copy(idx_hbm.at[pl.ds(c * _IDX_CHUNK, _IDX_CHUNK)], idx_v)

            def gather16(i, _):
                idx16 = idx_v[pl.ds(i * _L, _L)]
                col_v[pl.ds(i * _L, _L)] = plsc.load_gather(row_v, [idx16])
                return 0

            lax.fori_loop(0, _IDX_CHUNK // _L, gather16, 0)
            pltpu.sync_copy(col_v, out_hbm.at[d, pl.ds(c * _IDX_CHUNK, _IDX_CHUNK)])
            return 0

        lax.fori_loop(0, n_chunks, chunk, 0)

    return k


def kernel(domain_id, weight):
    (B,) = domain_id.shape
    V, D = weight.shape
    out_t = _make_gather(V, D, B)(weight.T, domain_id.astype(jnp.int32))
    return out_t.T


# speedup vs baseline: 2.4838x; 1.1082x over previous
import functools

import jax
import jax.numpy as jnp
from jax import lax
from jax.experimental import pallas as pl
from jax.experimental.pallas import tpu as pltpu
from jax.experimental.pallas import tpu_sc as plsc

_L = 16
_UNROLL = 8


@functools.cache
def _make_gather(V, D, B):
    info = plsc.get_sparse_core_info()
    NC, NS = info.num_cores, info.num_subcores
    NW = NC * NS
    assert D == NW, "one tile per embedding dim"
    chunk = 8192
    n_chunks = B // chunk
    step = _L * _UNROLL
    mesh = plsc.VectorSubcoreMesh(core_axis_name="c", subcore_axis_name="s")

    @functools.partial(
        pl.kernel,
        mesh=mesh,
        out_type=jax.ShapeDtypeStruct((D, B), jnp.float32),
        scratch_types=[
            pltpu.VMEM((V,), jnp.float32),
            pltpu.VMEM((chunk,), jnp.int32),
            pltpu.VMEM((chunk,), jnp.float32),
            pltpu.SemaphoreType.DMA,
            pltpu.SemaphoreType.DMA,
        ],
        compiler_params=pltpu.CompilerParams(
            use_tc_tiling_on_sc=True, needs_layout_passes=False
        ),
    )
    def k(wt_hbm, idx_hbm, out_hbm, row_v, idx_v, col_v, rsem, wsem):
        d = lax.axis_index("s") * NC + lax.axis_index("c")
        row_cp = pltpu.async_copy(wt_hbm.at[d], row_v, rsem)
        pltpu.sync_copy(idx_hbm.at[pl.ds(0, chunk)], idx_v)
        row_cp.wait()

        def gather_blk(i, _):
            off = i * step
            for u in range(_UNROLL):
                idx16 = idx_v[pl.ds(off + u * _L, _L)]
                col_v[pl.ds(off + u * _L, _L)] = plsc.load_gather(row_v, [idx16])
            return 0

        for c in range(n_chunks):
            lax.fori_loop(0, chunk // step, gather_blk, 0)
            wb = pltpu.async_copy(
                col_v, out_hbm.at[d, pl.ds(c * chunk, chunk)], wsem
            )
            if c + 1 < n_chunks:
                pltpu.sync_copy(
                    idx_hbm.at[pl.ds((c + 1) * chunk, chunk)], idx_v
                )
            wb.wait()

    return k


def kernel(domain_id, weight):
    (B,) = domain_id.shape
    V, D = weight.shape
    out_t = _make_gather(V, D, B)(weight.T, domain_id.astype(jnp.int32))
    return out_t.T
